# baseline (speedup 1.0000x reference)
import functools

import jax
import jax.numpy as jnp
from jax import lax
from jax.experimental import pallas as pl
from jax.experimental.pallas import tpu as pltpu
from jax.experimental.pallas import tpu_sc as plsc

_NW = 32
_IDX_CHUNK = 128


@functools.lru_cache(maxsize=None)
def _make_gather(B, V_u, D_u, V_i, D_i):
    b_per_w = B // _NW
    n_chunks = b_per_w // _IDX_CHUNK
    mesh = plsc.VectorSubcoreMesh(core_axis_name="c", subcore_axis_name="s")

    @functools.partial(
        pl.kernel,
        mesh=mesh,
        compiler_params=pltpu.CompilerParams(use_tc_tiling_on_sc=False),
        out_type=[
            jax.ShapeDtypeStruct((B, D_u), jnp.float32),
            jax.ShapeDtypeStruct((B, D_i), jnp.float32),
        ],
        scratch_types=[
            pltpu.VMEM((n_chunks, _IDX_CHUNK), jnp.int32),
            pltpu.VMEM((n_chunks, _IDX_CHUNK), jnp.int32),
            pltpu.VMEM((b_per_w, D_u), jnp.float32),
            pltpu.VMEM((b_per_w, D_i), jnp.float32),
            pltpu.SemaphoreType.DMA,
        ],
    )
    def gather_k(utab, itab, users, items, out_u, out_i,
                 uidx, iidx, urows, irows, sem):
        wid = lax.axis_index("s") * 2 + lax.axis_index("c")
        base = wid * b_per_w
        for j in range(n_chunks):
            pltpu.sync_copy(users.at[pl.ds(base + j * _IDX_CHUNK, _IDX_CHUNK)],
                            uidx.at[j])
            pltpu.sync_copy(items.at[pl.ds(base + j * _IDX_CHUNK, _IDX_CHUNK)],
                            iidx.at[j])
        copies = []
        for j in range(n_chunks):
            copies.append(pltpu.async_copy(
                utab.at[uidx.at[j]],
                urows.at[pl.ds(j * _IDX_CHUNK, _IDX_CHUNK)], sem))
            copies.append(pltpu.async_copy(
                itab.at[iidx.at[j]],
                irows.at[pl.ds(j * _IDX_CHUNK, _IDX_CHUNK)], sem))
        for c in copies:
            c.wait()
        pltpu.sync_copy(urows, out_u.at[pl.ds(base, b_per_w)])
        pltpu.sync_copy(irows, out_i.at[pl.ds(base, b_per_w)])

    return gather_k


def _mlp_body(xu_ref, xi_ref,
              wl1u, wl1i, bl1, wl2, bl2, wl3t, bl3,
              wr1u, wr1i, br1, wr2, br2, wr3t, br3,
              out_l, out_r):
    xu = xu_ref[...]
    xi = xi_ref[...]

    def head(w1u, w1i, b1, w2, b2, w3t, b3, out_ref):
        h = lax.dot(xu, w1u[...], preferred_element_type=jnp.float32)
        h = h + lax.dot(xi, w1i[...], preferred_element_type=jnp.float32)
        h = jnp.maximum(h + b1[...], 0.0)
        h = jnp.maximum(lax.dot(h, w2[...],
                                preferred_element_type=jnp.float32) + b2[...],
                        0.0)
        z = jnp.sum(h * w3t[...], axis=1) + b3[0, 0]
        out_ref[...] = 1.0 / (1.0 + jnp.exp(-z))

    head(wl1u, wl1i, bl1, wl2, bl2, wl3t, bl3, out_l)
    head(wr1u, wr1i, br1, wr2, br2, wr3t, br3, out_r)


@functools.lru_cache(maxsize=None)
def _make_mlp(B, D_u, D_i, H1, H2):
    BLK = 2048
    grid = B // BLK

    def w_spec(shape):
        return pl.BlockSpec(shape, lambda i: (0,) * len(shape))

    return pl.pallas_call(
        _mlp_body,
        grid=(grid,),
        in_specs=[
            pl.BlockSpec((BLK, D_u), lambda i: (i, 0)),
            pl.BlockSpec((BLK, D_i), lambda i: (i, 0)),
            w_spec((D_u, H1)), w_spec((D_i, H1)), w_spec((1, H1)),
            w_spec((H1, H2)), w_spec((1, H2)),
            w_spec((1, H2)), w_spec((1, 1)),
            w_spec((D_u, H1)), w_spec((D_i, H1)), w_spec((1, H1)),
            w_spec((H1, H2)), w_spec((1, H2)),
            w_spec((1, H2)), w_spec((1, 1)),
        ],
        out_specs=[
            pl.BlockSpec((BLK,), lambda i: (i,)),
            pl.BlockSpec((BLK,), lambda i: (i,)),
        ],
        out_shape=[
            jax.ShapeDtypeStruct((B,), jnp.float32),
            jax.ShapeDtypeStruct((B,), jnp.float32),
        ],
    )


def kernel(users, items, user_embedding, item_embedding,
           Wl1, bl1, Wl2, bl2, Wl3, bl3,
           Wr1, br1, Wr2, br2, Wr3, br3):
    B = users.shape[0]
    V_u, D_u = user_embedding.shape
    V_i, D_i = item_embedding.shape
    H1 = Wl1.shape[1]
    H2 = Wl2.shape[1]

    gather = _make_gather(B, V_u, D_u, V_i, D_i)
    xu, xi = gather(user_embedding, item_embedding, users, items)

    mlp = _make_mlp(B, D_u, D_i, H1, H2)
    likes, rec = mlp(
        xu, xi,
        Wl1[:D_u], Wl1[D_u:], bl1.reshape(1, H1),
        Wl2, bl2.reshape(1, H2), Wl3.reshape(1, H2), bl3.reshape(1, 1),
        Wr1[:D_u], Wr1[D_u:], br1.reshape(1, H1),
        Wr2, br2.reshape(1, H2), Wr3.reshape(1, H2), br3.reshape(1, 1),
    )
    return likes, rec


# speedup vs baseline: 1.6888x; 1.6888x over previous
import functools

import jax
import jax.numpy as jnp
from jax import lax
from jax.experimental import pallas as pl
from jax.experimental.pallas import tpu as pltpu
from jax.experimental.pallas import tpu_sc as plsc

_NW = 32
_IDX_CHUNK = 128


def _pack_u_body(x_ref, out_ref):
    x = x_ref[...]
    out_ref[...] = jnp.concatenate([x[:, :1024].T, x[:, 1024:].T], axis=1)


def _pack_i_body(x_ref, out_ref):
    x = x_ref[...]
    z = jnp.zeros((1024, 8), dtype=jnp.float32)
    parts = []
    for m in range(4):
        parts.append(x[:, m * 1024:(m + 1) * 1024].T)
        parts.append(z)
    out_ref[...] = jnp.concatenate(parts, axis=1)


@functools.lru_cache(maxsize=None)
def _make_pack_u(V, D):
    n = pl.cdiv(V, 2048)
    return pl.pallas_call(
        _pack_u_body,
        grid=(n,),
        in_specs=[pl.BlockSpec((D, 2048), lambda i: (0, i))],
        out_specs=pl.BlockSpec((1024, 128), lambda i: (i, 0)),
        out_shape=jax.ShapeDtypeStruct((n * 1024, 128), jnp.float32),
    )


@functools.lru_cache(maxsize=None)
def _make_pack_i(V, D):
    n = pl.cdiv(V, 4096)
    return pl.pallas_call(
        _pack_i_body,
        grid=(n,),
        in_specs=[pl.BlockSpec((D, 4096), lambda i: (0, i))],
        out_specs=pl.BlockSpec((1024, 128), lambda i: (i, 0)),
        out_shape=jax.ShapeDtypeStruct((n * 1024, 128), jnp.float32),
    )


@functools.lru_cache(maxsize=None)
def _make_gather(B):
    b_per_w = B // _NW
    n_chunks = b_per_w // _IDX_CHUNK
    n_half = n_chunks // 2
    rows_half = b_per_w // 2
    mesh = plsc.VectorSubcoreMesh(core_axis_name="c", subcore_axis_name="s")

    @functools.partial(
        pl.kernel,
        mesh=mesh,
        out_type=[
            jax.ShapeDtypeStruct((B, 128), jnp.float32),
            jax.ShapeDtypeStruct((B, 128), jnp.float32),
        ],
        scratch_types=[
            pltpu.VMEM((n_chunks, _IDX_CHUNK), jnp.int32),
            pltpu.VMEM((n_chunks, _IDX_CHUNK), jnp.int32),
            pltpu.VMEM((rows_half, 128), jnp.float32),
            pltpu.VMEM((rows_half, 128), jnp.float32),
            pltpu.SemaphoreType.DMA,
        ],
    )
    def gather_k(su, si, users, items, out_u, out_i,
                 uidx, iidx, urows, irows, sem):
        wid = lax.axis_index("s") * 2 + lax.axis_index("c")
        base = wid * b_per_w
        for j in range(n_chunks):
            pltpu.sync_copy(users.at[pl.ds(base + j * _IDX_CHUNK, _IDX_CHUNK)],
                            uidx.at[j])
            pltpu.sync_copy(items.at[pl.ds(base + j * _IDX_CHUNK, _IDX_CHUNK)],
                            iidx.at[j])
        for j in range(n_chunks):
            for k in range(_IDX_CHUNK // 16):
                s = pl.ds(k * 16, 16)
                u = uidx[j, s]
                uidx[j, s] = ((u >> 11) << 10) | (u & 1023)
                it = iidx[j, s]
                iidx[j, s] = ((it >> 12) << 10) | (it & 1023)
        for h in range(2):
            copies = []
            for j2 in range(n_half):
                j = h * n_half + j2
                dst = pl.ds(j2 * _IDX_CHUNK, _IDX_CHUNK)
                copies.append(pltpu.async_copy(su.at[uidx.at[j]],
                                               urows.at[dst], sem))
                copies.append(pltpu.async_copy(si.at[iidx.at[j]],
                                               irows.at[dst], sem))
            for c in copies:
                c.wait()
            out = pl.ds(base + h * rows_half, rows_half)
            pltpu.sync_copy(urows, out_u.at[out])
            pltpu.sync_copy(irows, out_i.at[out])

    return gather_k


def _mlp_body(uraw_ref, iraw_ref, users_ref, items_ref,
              wl1u, wl1i, bl1, wl2, bl2, wl3t, bl3,
              wr1u, wr1i, br1, wr2, br2, wr3t, br3,
              out_l, out_r):
    u = users_ref[...]
    it = items_ref[...]
    uraw = uraw_ref[...]
    iraw = iraw_ref[...]
    hi = ((u >> 10) & 1)[:, None]
    xu = jnp.where(hi == 1, uraw[:, 64:128], uraw[:, 0:64])
    m = ((it >> 10) & 3)[:, None]
    xi = jnp.where(
        m == 0, iraw[:, 0:24],
        jnp.where(m == 1, iraw[:, 32:56],
                  jnp.where(m == 2, iraw[:, 64:88], iraw[:, 96:120])))

    def head(w1u, w1i, b1, w2, b2, w3t, b3, out_ref):
        h = lax.dot(xu, w1u[...], preferred_element_type=jnp.float32)
        h = h + lax.dot(xi, w1i[...], preferred_element_type=jnp.float32)
        h = jnp.maximum(h + b1[...], 0.0)
        h = jnp.maximum(lax.dot(h, w2[...],
                                preferred_element_type=jnp.float32) + b2[...],
                        0.0)
        z = jnp.sum(h * w3t[...], axis=1) + b3[0, 0]
        out_ref[...] = 1.0 / (1.0 + jnp.exp(-z))

    head(wl1u, wl1i, bl1, wl2, bl2, wl3t, bl3, out_l)
    head(wr1u, wr1i, br1, wr2, br2, wr3t, br3, out_r)


@functools.lru_cache(maxsize=None)
def _make_mlp(B, D_u, D_i, H1, H2):
    BLK = 2048
    grid = B // BLK

    def w_spec(shape):
        return pl.BlockSpec(shape, lambda i: (0,) * len(shape))

    return pl.pallas_call(
        _mlp_body,
        grid=(grid,),
        in_specs=[
            pl.BlockSpec((BLK, 128), lambda i: (i, 0)),
            pl.BlockSpec((BLK, 128), lambda i: (i, 0)),
            pl.BlockSpec((BLK,), lambda i: (i,)),
            pl.BlockSpec((BLK,), lambda i: (i,)),
            w_spec((D_u, H1)), w_spec((D_i, H1)), w_spec((1, H1)),
            w_spec((H1, H2)), w_spec((1, H2)),
            w_spec((1, H2)), w_spec((1, 1)),
            w_spec((D_u, H1)), w_spec((D_i, H1)), w_spec((1, H1)),
            w_spec((H1, H2)), w_spec((1, H2)),
            w_spec((1, H2)), w_spec((1, 1)),
        ],
        out_specs=[
            pl.BlockSpec((BLK,), lambda i: (i,)),
            pl.BlockSpec((BLK,), lambda i: (i,)),
        ],
        out_shape=[
            jax.ShapeDtypeStruct((B,), jnp.float32),
            jax.ShapeDtypeStruct((B,), jnp.float32),
        ],
    )


def kernel(users, items, user_embedding, item_embedding,
           Wl1, bl1, Wl2, bl2, Wl3, bl3,
           Wr1, br1, Wr2, br2, Wr3, br3):
    B = users.shape[0]
    V_u, D_u = user_embedding.shape
    V_i, D_i = item_embedding.shape
    H1 = Wl1.shape[1]
    H2 = Wl2.shape[1]

    su = _make_pack_u(V_u, D_u)(user_embedding.T)
    si = _make_pack_i(V_i, D_i)(item_embedding.T)

    uraw, iraw = _make_gather(B)(su, si, users, items)

    mlp = _make_mlp(B, D_u, D_i, H1, H2)
    likes, rec = mlp(
        uraw, iraw, users, items,
        Wl1[:D_u], Wl1[D_u:], bl1.reshape(1, H1),
        Wl2, bl2.reshape(1, H2), Wl3.reshape(1, H2), bl3.reshape(1, 1),
        Wr1[:D_u], Wr1[D_u:], br1.reshape(1, H1),
        Wr2, br2.reshape(1, H2), Wr3.reshape(1, H2), br3.reshape(1, 1),
    )
    return likes, rec


# speedup vs baseline: 2.0080x; 1.1890x over previous
import functools

import jax
import jax.numpy as jnp
from jax import lax
from jax.experimental import pallas as pl
from jax.experimental.pallas import tpu as pltpu
from jax.experimental.pallas import tpu_sc as plsc

_NW = 32
_IDX_CHUNK = 128


def _round_bits(x):
    return lax.bitcast_convert_type(
        x.astype(jnp.bfloat16).astype(jnp.float32), jnp.uint32)


def _pack_u_body(x_ref, out_ref):
    xb = _round_bits(x_ref[...])
    w = xb[0:32] | (xb[32:64] >> 16)
    wt = lax.bitcast_convert_type(w, jnp.float32).T
    out_ref[...] = jnp.concatenate(
        [wt[s * 1024:(s + 1) * 1024] for s in range(4)], axis=1)


def _pack_i_body(x_ref, out_ref):
    xb = _round_bits(x_ref[...])
    lo = jnp.concatenate(
        [xb[16:24] >> 16, jnp.zeros((8, xb.shape[1]), jnp.uint32)], axis=0)
    w = xb[0:16] | lo
    wt = lax.bitcast_convert_type(w, jnp.float32).T
    out_ref[...] = jnp.concatenate(
        [wt[s * 1024:(s + 1) * 1024] for s in range(8)], axis=1)


@functools.lru_cache(maxsize=None)
def _make_pack(V, D, cols_per_block, body):
    n = pl.cdiv(V, cols_per_block)
    return pl.pallas_call(
        body,
        grid=(n,),
        in_specs=[pl.BlockSpec((D, cols_per_block), lambda i: (0, i))],
        out_specs=pl.BlockSpec((1024, 128), lambda i: (i, 0)),
        out_shape=jax.ShapeDtypeStruct((n * 1024, 128), jnp.float32),
    )


@functools.lru_cache(maxsize=None)
def _make_gather(B):
    b_per_w = B // _NW
    n_chunks = b_per_w // _IDX_CHUNK
    n_half = n_chunks // 2
    rows_half = b_per_w // 2
    mesh = plsc.VectorSubcoreMesh(core_axis_name="c", subcore_axis_name="s")

    @functools.partial(
        pl.kernel,
        mesh=mesh,
        out_type=[
            jax.ShapeDtypeStruct((B, 128), jnp.float32),
            jax.ShapeDtypeStruct((B, 128), jnp.float32),
        ],
        scratch_types=[
            pltpu.VMEM((n_chunks, _IDX_CHUNK), jnp.int32),
            pltpu.VMEM((n_chunks, _IDX_CHUNK), jnp.int32),
            pltpu.VMEM((rows_half, 128), jnp.float32),
            pltpu.VMEM((rows_half, 128), jnp.float32),
            pltpu.SemaphoreType.DMA,
        ],
    )
    def gather_k(su, si, users, items, out_u, out_i,
                 uidx, iidx, urows, irows, sem):
        wid = lax.axis_index("s") * 2 + lax.axis_index("c")
        base = wid * b_per_w
        for j in range(n_chunks):
            pltpu.sync_copy(users.at[pl.ds(base + j * _IDX_CHUNK, _IDX_CHUNK)],
                            uidx.at[j])
            pltpu.sync_copy(items.at[pl.ds(base + j * _IDX_CHUNK, _IDX_CHUNK)],
                            iidx.at[j])
        for j in range(n_chunks):
            for k in range(_IDX_CHUNK // 16):
                s = pl.ds(k * 16, 16)
                u = uidx[j, s]
                uidx[j, s] = ((u >> 12) << 10) | (u & 1023)
                it = iidx[j, s]
                iidx[j, s] = ((it >> 13) << 10) | (it & 1023)
        for h in range(2):
            copies = []
            for j2 in range(n_half):
                j = h * n_half + j2
                dst = pl.ds(j2 * _IDX_CHUNK, _IDX_CHUNK)
                copies.append(pltpu.async_copy(su.at[uidx.at[j]],
                                               urows.at[dst], sem))
                copies.append(pltpu.async_copy(si.at[iidx.at[j]],
                                               irows.at[dst], sem))
            for c in copies:
                c.wait()
            out = pl.ds(base + h * rows_half, rows_half)
            pltpu.sync_copy(urows, out_u.at[out])
            pltpu.sync_copy(irows, out_i.at[out])

    return gather_k


def _mlp_body(uraw_ref, iraw_ref, users_ref, items_ref,
              wl1u, wl1i, bl1, wl2, bl2, wl3t, bl3,
              wr1u, wr1i, br1, wr2, br2, wr3t, br3,
              out_l, out_r):
    u = users_ref[...]
    it = items_ref[...]

    def unpack(words):
        hi = lax.bitcast_convert_type(words & jnp.uint32(0xFFFF0000),
                                      jnp.float32)
        lo = lax.bitcast_convert_type(words << 16, jnp.float32)
        return jnp.concatenate([hi, lo], axis=1)

    ubits = lax.bitcast_convert_type(uraw_ref[...], jnp.uint32)
    ibits = lax.bitcast_convert_type(iraw_ref[...], jnp.uint32)
    s_u = ((u >> 10) & 3)[:, None]
    uw = jnp.where(
        s_u == 0, ubits[:, 0:32],
        jnp.where(s_u == 1, ubits[:, 32:64],
                  jnp.where(s_u == 2, ubits[:, 64:96], ubits[:, 96:128])))
    xu = unpack(uw)
    s_i = ((it >> 10) & 7)[:, None]
    iw = ibits[:, 112:128]
    for s in range(6, -1, -1):
        iw = jnp.where(s_i == s, ibits[:, 16 * s:16 * s + 16], iw)
    xi = unpack(iw)[:, 0:24]

    def head(w1u, w1i, b1, w2, b2, w3t, b3, out_ref):
        h = lax.dot(xu, w1u[...], preferred_element_type=jnp.float32)
        h = h + lax.dot(xi, w1i[...], preferred_element_type=jnp.float32)
        h = jnp.maximum(h + b1[...], 0.0)
        h = jnp.maximum(lax.dot(h, w2[...],
                                preferred_element_type=jnp.float32) + b2[...],
                        0.0)
        z = jnp.sum(h * w3t[...], axis=1) + b3[0, 0]
        out_ref[...] = 1.0 / (1.0 + jnp.exp(-z))

    head(wl1u, wl1i, bl1, wl2, bl2, wl3t, bl3, out_l)
    head(wr1u, wr1i, br1, wr2, br2, wr3t, br3, out_r)


@functools.lru_cache(maxsize=None)
def _make_mlp(B, D_u, D_i, H1, H2):
    BLK = 2048
    grid = B // BLK

    def w_spec(shape):
        return pl.BlockSpec(shape, lambda i: (0,) * len(shape))

    return pl.pallas_call(
        _mlp_body,
        grid=(grid,),
        in_specs=[
            pl.BlockSpec((BLK, 128), lambda i: (i, 0)),
            pl.BlockSpec((BLK, 128), lambda i: (i, 0)),
            pl.BlockSpec((BLK,), lambda i: (i,)),
            pl.BlockSpec((BLK,), lambda i: (i,)),
            w_spec((D_u, H1)), w_spec((D_i, H1)), w_spec((1, H1)),
            w_spec((H1, H2)), w_spec((1, H2)),
            w_spec((1, H2)), w_spec((1, 1)),
            w_spec((D_u, H1)), w_spec((D_i, H1)), w_spec((1, H1)),
            w_spec((H1, H2)), w_spec((1, H2)),
            w_spec((1, H2)), w_spec((1, 1)),
        ],
        out_specs=[
            pl.BlockSpec((BLK,), lambda i: (i,)),
            pl.BlockSpec((BLK,), lambda i: (i,)),
        ],
        out_shape=[
            jax.ShapeDtypeStruct((B,), jnp.float32),
            jax.ShapeDtypeStruct((B,), jnp.float32),
        ],
    )


def kernel(users, items, user_embedding, item_embedding,
           Wl1, bl1, Wl2, bl2, Wl3, bl3,
           Wr1, br1, Wr2, br2, Wr3, br3):
    B = users.shape[0]
    V_u, D_u = user_embedding.shape
    V_i, D_i = item_embedding.shape
    H1 = Wl1.shape[1]
    H2 = Wl2.shape[1]

    su = _make_pack(V_u, D_u, 4096, _pack_u_body)(user_embedding.T)
    si = _make_pack(V_i, D_i, 8192, _pack_i_body)(item_embedding.T)

    uraw, iraw = _make_gather(B)(su, si, users, items)

    mlp = _make_mlp(B, D_u, D_i, H1, H2)
    likes, rec = mlp(
        uraw, iraw, users, items,
        Wl1[:D_u], Wl1[D_u:], bl1.reshape(1, H1),
        Wl2, bl2.reshape(1, H2), Wl3.reshape(1, H2), bl3.reshape(1, 1),
        Wr1[:D_u], Wr1[D_u:], br1.reshape(1, H1),
        Wr2, br2.reshape(1, H2), Wr3.reshape(1, H2), br3.reshape(1, 1),
    )
    return likes, rec


# speedup vs baseline: 2.2362x; 1.1136x over previous
import functools

import jax
import jax.numpy as jnp
from jax import lax
from jax.experimental import pallas as pl
from jax.experimental.pallas import tpu as pltpu
from jax.experimental.pallas import tpu_sc as plsc

_NW = 32
_IDX_CHUNK = 128


def _round_bits(x):
    return lax.bitcast_convert_type(
        x.astype(jnp.bfloat16).astype(jnp.float32), jnp.uint32)


_ROWS = 2048


def _pack_u_body(x_ref, out_ref):
    xb = _round_bits(x_ref[...])
    w = xb[0:32] | (xb[32:64] >> 16)
    wt = lax.bitcast_convert_type(w, jnp.float32).T
    out_ref[...] = jnp.concatenate(
        [wt[s * _ROWS:(s + 1) * _ROWS] for s in range(4)], axis=1)


def _pack_i_body(x_ref, out_ref):
    xb = _round_bits(x_ref[...])
    lo = jnp.concatenate(
        [xb[16:24] >> 16, jnp.zeros((8, xb.shape[1]), jnp.uint32)], axis=0)
    w = xb[0:16] | lo
    wt = lax.bitcast_convert_type(w, jnp.float32).T
    out_ref[...] = jnp.concatenate(
        [wt[s * _ROWS:(s + 1) * _ROWS] for s in range(8)], axis=1)


@functools.lru_cache(maxsize=None)
def _make_pack(V, D, cols_per_block, body):
    n = pl.cdiv(V, cols_per_block)
    return pl.pallas_call(
        body,
        grid=(n,),
        in_specs=[pl.BlockSpec((D, cols_per_block), lambda i: (0, i))],
        out_specs=pl.BlockSpec((_ROWS, 128), lambda i: (i, 0)),
        out_shape=jax.ShapeDtypeStruct((n * _ROWS, 128), jnp.float32),
    )


@functools.lru_cache(maxsize=None)
def _make_gather(B):
    b_per_w = B // _NW
    n_chunks = b_per_w // _IDX_CHUNK
    n_half = n_chunks // 2
    rows_half = b_per_w // 2
    mesh = plsc.VectorSubcoreMesh(core_axis_name="c", subcore_axis_name="s")

    @functools.partial(
        pl.kernel,
        mesh=mesh,
        out_type=[
            jax.ShapeDtypeStruct((B, 128), jnp.float32),
            jax.ShapeDtypeStruct((B, 128), jnp.float32),
        ],
        scratch_types=[
            pltpu.VMEM((n_chunks, _IDX_CHUNK), jnp.int32),
            pltpu.VMEM((n_chunks, _IDX_CHUNK), jnp.int32),
            pltpu.VMEM((rows_half, 128), jnp.float32),
            pltpu.VMEM((rows_half, 128), jnp.float32),
            pltpu.SemaphoreType.DMA,
        ],
    )
    def gather_k(su, si, users, items, out_u, out_i,
                 uidx, iidx, urows, irows, sem):
        wid = lax.axis_index("s") * 2 + lax.axis_index("c")
        base = wid * b_per_w
        for j in range(n_chunks):
            pltpu.sync_copy(users.at[pl.ds(base + j * _IDX_CHUNK, _IDX_CHUNK)],
                            uidx.at[j])
            pltpu.sync_copy(items.at[pl.ds(base + j * _IDX_CHUNK, _IDX_CHUNK)],
                            iidx.at[j])
        for j in range(n_chunks):
            for k in range(_IDX_CHUNK // 16):
                s = pl.ds(k * 16, 16)
                u = uidx[j, s]
                uidx[j, s] = ((u >> 13) << 11) | (u & 2047)
                it = iidx[j, s]
                iidx[j, s] = ((it >> 14) << 11) | (it & 2047)
        for h in range(2):
            copies = []
            for j2 in range(n_half):
                j = h * n_half + j2
                dst = pl.ds(j2 * _IDX_CHUNK, _IDX_CHUNK)
                copies.append(pltpu.async_copy(su.at[uidx.at[j]],
                                               urows.at[dst], sem))
                copies.append(pltpu.async_copy(si.at[iidx.at[j]],
                                               irows.at[dst], sem))
            for c in copies:
                c.wait()
            out = pl.ds(base + h * rows_half, rows_half)
            pltpu.sync_copy(urows, out_u.at[out])
            pltpu.sync_copy(irows, out_i.at[out])

    return gather_k


def _mlp_body(uraw_ref, iraw_ref, users_ref, items_ref,
              wl1u, wl1i, bl1, wl2, bl2, wl3t, bl3,
              wr1u, wr1i, br1, wr2, br2, wr3t, br3,
              out_l, out_r):
    u = users_ref[...]
    it = items_ref[...]

    def unpack(words):
        hi = lax.bitcast_convert_type(words & jnp.uint32(0xFFFF0000),
                                      jnp.float32)
        lo = lax.bitcast_convert_type(words << 16, jnp.float32)
        return jnp.concatenate([hi, lo], axis=1)

    ubits = lax.bitcast_convert_type(uraw_ref[...], jnp.uint32)
    ibits = lax.bitcast_convert_type(iraw_ref[...], jnp.uint32)
    s_u = ((u >> 11) & 3)[:, None]
    uw = jnp.where(
        s_u == 0, ubits[:, 0:32],
        jnp.where(s_u == 1, ubits[:, 32:64],
                  jnp.where(s_u == 2, ubits[:, 64:96], ubits[:, 96:128])))
    xu = unpack(uw)
    s_i = ((it >> 11) & 7)[:, None]
    iw = ibits[:, 112:128]
    for s in range(6, -1, -1):
        iw = jnp.where(s_i == s, ibits[:, 16 * s:16 * s + 16], iw)
    xi = unpack(iw)[:, 0:24]

    def head(w1u, w1i, b1, w2, b2, w3, b3, out_ref):
        h = lax.dot(xu, w1u[...], preferred_element_type=jnp.float32)
        h = h + lax.dot(xi, w1i[...], preferred_element_type=jnp.float32)
        h = jnp.maximum(h + b1[...], 0.0)
        h = jnp.maximum(lax.dot(h, w2[...],
                                preferred_element_type=jnp.float32) + b2[...],
                        0.0)
        z = lax.dot(h, w3[...], preferred_element_type=jnp.float32) + b3[...]
        out_ref[...] = 1.0 / (1.0 + jnp.exp(-z))

    head(wl1u, wl1i, bl1, wl2, bl2, wl3t, bl3, out_l)
    head(wr1u, wr1i, br1, wr2, br2, wr3t, br3, out_r)


@functools.lru_cache(maxsize=None)
def _make_mlp(B, D_u, D_i, H1, H2):
    BLK = 2048
    grid = B // BLK

    def w_spec(shape):
        return pl.BlockSpec(shape, lambda i: (0,) * len(shape))

    return pl.pallas_call(
        _mlp_body,
        grid=(grid,),
        in_specs=[
            pl.BlockSpec((BLK, 128), lambda i: (i, 0)),
            pl.BlockSpec((BLK, 128), lambda i: (i, 0)),
            pl.BlockSpec((BLK,), lambda i: (i,)),
            pl.BlockSpec((BLK,), lambda i: (i,)),
            w_spec((D_u, H1)), w_spec((D_i, H1)), w_spec((1, H1)),
            w_spec((H1, H2)), w_spec((1, H2)),
            w_spec((H2, 1)), w_spec((1, 1)),
            w_spec((D_u, H1)), w_spec((D_i, H1)), w_spec((1, H1)),
            w_spec((H1, H2)), w_spec((1, H2)),
            w_spec((H2, 1)), w_spec((1, 1)),
        ],
        out_specs=[
            pl.BlockSpec((BLK, 1), lambda i: (i, 0)),
            pl.BlockSpec((BLK, 1), lambda i: (i, 0)),
        ],
        out_shape=[
            jax.ShapeDtypeStruct((B, 1), jnp.float32),
            jax.ShapeDtypeStruct((B, 1), jnp.float32),
        ],
    )


def kernel(users, items, user_embedding, item_embedding,
           Wl1, bl1, Wl2, bl2, Wl3, bl3,
           Wr1, br1, Wr2, br2, Wr3, br3):
    B = users.shape[0]
    V_u, D_u = user_embedding.shape
    V_i, D_i = item_embedding.shape
    H1 = Wl1.shape[1]
    H2 = Wl2.shape[1]

    su = _make_pack(V_u, D_u, 4 * _ROWS, _pack_u_body)(user_embedding.T)
    si = _make_pack(V_i, D_i, 8 * _ROWS, _pack_i_body)(item_embedding.T)

    uraw, iraw = _make_gather(B)(su, si, users, items)

    mlp = _make_mlp(B, D_u, D_i, H1, H2)
    likes, rec = mlp(
        uraw, iraw, users, items,
        Wl1[:D_u], Wl1[D_u:], bl1.reshape(1, H1),
        Wl2, bl2.reshape(1, H2), Wl3, bl3.reshape(1, 1),
        Wr1[:D_u], Wr1[D_u:], br1.reshape(1, H1),
        Wr2, br2.reshape(1, H2), Wr3, br3.reshape(1, 1),
    )
    return jnp.squeeze(likes, -1), jnp.squeeze(rec, -1)


# speedup vs baseline: 4.2518x; 1.9014x over previous
import functools

import jax
import jax.numpy as jnp
from jax import lax
from jax.experimental import pallas as pl
from jax.experimental.pallas import tpu as pltpu
from jax.experimental.pallas import tpu_sc as plsc

_NW = 32
_IDX_CHUNK = 128


def _round_bits(x):
    return lax.bitcast_convert_type(
        x.astype(jnp.bfloat16).astype(jnp.float32), jnp.uint32)


_ROWS = 2048


def _pack_u_body(x_ref, out_ref):
    xb = _round_bits(x_ref[...])
    w = xb[0:32] | (xb[32:64] >> 16)
    w2 = jnp.concatenate(
        [w[:, s * _ROWS:(s + 1) * _ROWS] for s in range(4)], axis=0)
    out_ref[...] = lax.bitcast_convert_type(w2, jnp.float32).T


def _pack_i_body(x_ref, out_ref):
    xb = _round_bits(x_ref[...])
    lo = jnp.concatenate(
        [xb[16:24] >> 16, jnp.zeros((8, xb.shape[1]), jnp.uint32)], axis=0)
    w = xb[0:16] | lo
    w2 = jnp.concatenate(
        [w[:, s * _ROWS:(s + 1) * _ROWS] for s in range(8)], axis=0)
    out_ref[...] = lax.bitcast_convert_type(w2, jnp.float32).T


@functools.lru_cache(maxsize=None)
def _make_pack(V, D, cols_per_block, body):
    n = pl.cdiv(V, cols_per_block)
    return pl.pallas_call(
        body,
        grid=(n,),
        in_specs=[pl.BlockSpec((D, cols_per_block), lambda i: (0, i))],
        out_specs=pl.BlockSpec((_ROWS, 128), lambda i: (i, 0)),
        out_shape=jax.ShapeDtypeStruct((n * _ROWS, 128), jnp.float32),
    )


@functools.lru_cache(maxsize=None)
def _make_gather(B):
    b_per_w = B // _NW
    n_chunks = b_per_w // _IDX_CHUNK
    n_half = n_chunks // 2
    rows_half = b_per_w // 2
    mesh = plsc.VectorSubcoreMesh(core_axis_name="c", subcore_axis_name="s")

    @functools.partial(
        pl.kernel,
        mesh=mesh,
        out_type=[
            jax.ShapeDtypeStruct((B, 128), jnp.float32),
            jax.ShapeDtypeStruct((B, 128), jnp.float32),
        ],
        scratch_types=[
            pltpu.VMEM((n_chunks, _IDX_CHUNK), jnp.int32),
            pltpu.VMEM((n_chunks, _IDX_CHUNK), jnp.int32),
            pltpu.VMEM((rows_half, 128), jnp.float32),
            pltpu.VMEM((rows_half, 128), jnp.float32),
            pltpu.SemaphoreType.DMA,
        ],
    )
    def gather_k(su, si, users, items, out_u, out_i,
                 uidx, iidx, urows, irows, sem):
        wid = lax.axis_index("s") * 2 + lax.axis_index("c")
        base = wid * b_per_w
        for j in range(n_chunks):
            pltpu.sync_copy(users.at[pl.ds(base + j * _IDX_CHUNK, _IDX_CHUNK)],
                            uidx.at[j])
            pltpu.sync_copy(items.at[pl.ds(base + j * _IDX_CHUNK, _IDX_CHUNK)],
                            iidx.at[j])
        for j in range(n_chunks):
            for k in range(_IDX_CHUNK // 16):
                s = pl.ds(k * 16, 16)
                u = uidx[j, s]
                uidx[j, s] = ((u >> 13) << 11) | (u & 2047)
                it = iidx[j, s]
                iidx[j, s] = ((it >> 14) << 11) | (it & 2047)
        for h in range(2):
            copies = []
            for j2 in range(n_half):
                j = h * n_half + j2
                dst = pl.ds(j2 * _IDX_CHUNK, _IDX_CHUNK)
                copies.append(pltpu.async_copy(su.at[uidx.at[j]],
                                               urows.at[dst], sem))
                copies.append(pltpu.async_copy(si.at[iidx.at[j]],
                                               irows.at[dst], sem))
            for c in copies:
                c.wait()
            out = pl.ds(base + h * rows_half, rows_half)
            pltpu.sync_copy(urows, out_u.at[out])
            pltpu.sync_copy(irows, out_i.at[out])

    return gather_k


def _mlp_body(uraw_ref, iraw_ref, users_ref, items_ref,
              wl1u, wl1i, bl1, wl2, bl2, wl3t, bl3,
              wr1u, wr1i, br1, wr2, br2, wr3t, br3,
              out_l, out_r):
    u = users_ref[...]
    it = items_ref[...]

    def unpack(words):
        hi = lax.bitcast_convert_type(words & jnp.uint32(0xFFFF0000),
                                      jnp.float32)
        lo = lax.bitcast_convert_type(words << 16, jnp.float32)
        return jnp.concatenate([hi, lo], axis=1)

    ubits = lax.bitcast_convert_type(uraw_ref[...], jnp.uint32)
    ibits = lax.bitcast_convert_type(iraw_ref[...], jnp.uint32)
    s_u = ((u >> 11) & 3)[:, None]
    uw = jnp.where(
        s_u == 0, ubits[:, 0:32],
        jnp.where(s_u == 1, ubits[:, 32:64],
                  jnp.where(s_u == 2, ubits[:, 64:96], ubits[:, 96:128])))
    xu = unpack(uw)
    s_i = ((it >> 11) & 7)[:, None]
    iw = ibits[:, 112:128]
    for s in range(6, -1, -1):
        iw = jnp.where(s_i == s, ibits[:, 16 * s:16 * s + 16], iw)
    xi = unpack(iw)[:, 0:24]

    def head(w1u, w1i, b1, w2, b2, w3, b3, out_ref):
        h = lax.dot(xu, w1u[...], preferred_element_type=jnp.float32)
        h = h + lax.dot(xi, w1i[...], preferred_element_type=jnp.float32)
        h = jnp.maximum(h + b1[...], 0.0)
        h = jnp.maximum(lax.dot(h, w2[...],
                                preferred_element_type=jnp.float32) + b2[...],
                        0.0)
        z = lax.dot(h, w3[...], preferred_element_type=jnp.float32) + b3[...]
        out_ref[...] = 1.0 / (1.0 + jnp.exp(-z))

    head(wl1u, wl1i, bl1, wl2, bl2, wl3t, bl3, out_l)
    head(wr1u, wr1i, br1, wr2, br2, wr3t, br3, out_r)


@functools.lru_cache(maxsize=None)
def _make_mlp(B, D_u, D_i, H1, H2):
    BLK = 2048
    grid = B // BLK

    def w_spec(shape):
        return pl.BlockSpec(shape, lambda i: (0,) * len(shape))

    return pl.pallas_call(
        _mlp_body,
        grid=(grid,),
        in_specs=[
            pl.BlockSpec((BLK, 128), lambda i: (i, 0)),
            pl.BlockSpec((BLK, 128), lambda i: (i, 0)),
            pl.BlockSpec((BLK,), lambda i: (i,)),
            pl.BlockSpec((BLK,), lambda i: (i,)),
            w_spec((D_u, H1)), w_spec((D_i, H1)), w_spec((1, H1)),
            w_spec((H1, H2)), w_spec((1, H2)),
            w_spec((H2, 1)), w_spec((1, 1)),
            w_spec((D_u, H1)), w_spec((D_i, H1)), w_spec((1, H1)),
            w_spec((H1, H2)), w_spec((1, H2)),
            w_spec((H2, 1)), w_spec((1, 1)),
        ],
        out_specs=[
            pl.BlockSpec((BLK, 1), lambda i: (i, 0)),
            pl.BlockSpec((BLK, 1), lambda i: (i, 0)),
        ],
        out_shape=[
            jax.ShapeDtypeStruct((B, 1), jnp.float32),
            jax.ShapeDtypeStruct((B, 1), jnp.float32),
        ],
    )


def kernel(users, items, user_embedding, item_embedding,
           Wl1, bl1, Wl2, bl2, Wl3, bl3,
           Wr1, br1, Wr2, br2, Wr3, br3):
    B = users.shape[0]
    V_u, D_u = user_embedding.shape
    V_i, D_i = item_embedding.shape
    H1 = Wl1.shape[1]
    H2 = Wl2.shape[1]

    su = _make_pack(V_u, D_u, 4 * _ROWS, _pack_u_body)(user_embedding.T)
    si = _make_pack(V_i, D_i, 8 * _ROWS, _pack_i_body)(item_embedding.T)

    uraw, iraw = _make_gather(B)(su, si, users, items)

    mlp = _make_mlp(B, D_u, D_i, H1, H2)
    likes, rec = mlp(
        uraw, iraw, users, items,
        Wl1[:D_u], Wl1[D_u:], bl1.reshape(1, H1),
        Wl2, bl2.reshape(1, H2), Wl3, bl3.reshape(1, 1),
        Wr1[:D_u], Wr1[D_u:], br1.reshape(1, H1),
        Wr2, br2.reshape(1, H2), Wr3, br3.reshape(1, 1),
    )
    return jnp.squeeze(likes, -1), jnp.squeeze(rec, -1)


# speedup vs baseline: 4.2882x; 1.0086x over previous
import functools

import jax
import jax.numpy as jnp
from jax import lax
from jax.experimental import pallas as pl
from jax.experimental.pallas import tpu as pltpu
from jax.experimental.pallas import tpu_sc as plsc

_NW = 32
_IDX_CHUNK = 128


def _round_bits(x):
    return lax.bitcast_convert_type(
        x.astype(jnp.bfloat16).astype(jnp.float32), jnp.uint32)


_ROWS = 2048


def _pack_u_body(x_ref, out_ref):
    xb = _round_bits(x_ref[...])
    w = xb[0:32] | (xb[32:64] >> 16)
    w2 = jnp.concatenate(
        [w[:, s * _ROWS:(s + 1) * _ROWS] for s in range(4)], axis=0)
    out_ref[...] = lax.bitcast_convert_type(w2, jnp.float32).T


def _pack_i_body(x_ref, out_ref):
    xb = _round_bits(x_ref[...])
    lo = jnp.concatenate(
        [xb[16:24] >> 16, jnp.zeros((8, xb.shape[1]), jnp.uint32)], axis=0)
    w = xb[0:16] | lo
    w2 = jnp.concatenate(
        [w[:, s * _ROWS:(s + 1) * _ROWS] for s in range(8)], axis=0)
    out_ref[...] = lax.bitcast_convert_type(w2, jnp.float32).T


@functools.lru_cache(maxsize=None)
def _make_pack(V, D, cols_per_block, body):
    n = pl.cdiv(V, cols_per_block)
    return pl.pallas_call(
        body,
        grid=(n,),
        in_specs=[pl.BlockSpec((D, cols_per_block), lambda i: (0, i))],
        out_specs=pl.BlockSpec((_ROWS, 128), lambda i: (i, 0)),
        out_shape=jax.ShapeDtypeStruct((n * _ROWS, 128), jnp.float32),
    )


@functools.lru_cache(maxsize=None)
def _make_gather(B, row_shift):
    b_per_w = B // _NW
    n_chunks = b_per_w // _IDX_CHUNK
    mesh = plsc.VectorSubcoreMesh(core_axis_name="c", subcore_axis_name="s")

    @functools.partial(
        pl.kernel,
        mesh=mesh,
        out_type=jax.ShapeDtypeStruct((B, 128), jnp.float32),
        scratch_types=[
            pltpu.VMEM((n_chunks, _IDX_CHUNK), jnp.int32),
            pltpu.VMEM((b_per_w, 128), jnp.float32),
            pltpu.SemaphoreType.DMA,
        ],
    )
    def gather_k(tab, idx_hbm, out, idx, rows, sem):
        wid = lax.axis_index("s") * 2 + lax.axis_index("c")
        base = wid * b_per_w
        for j in range(n_chunks):
            pltpu.sync_copy(idx_hbm.at[pl.ds(base + j * _IDX_CHUNK,
                                             _IDX_CHUNK)],
                            idx.at[j])
        for j in range(n_chunks):
            for k in range(_IDX_CHUNK // 16):
                s = pl.ds(k * 16, 16)
                v = idx[j, s]
                idx[j, s] = ((v >> row_shift) << 11) | (v & 2047)
        copies = []
        for j in range(n_chunks):
            copies.append(pltpu.async_copy(
                tab.at[idx.at[j]],
                rows.at[pl.ds(j * _IDX_CHUNK, _IDX_CHUNK)], sem))
        for c in copies:
            c.wait()
        pltpu.sync_copy(rows, out.at[pl.ds(base, b_per_w)])

    return gather_k


def _mlp_body(uraw_ref, iraw_ref, users_ref, items_ref,
              wl1u, wl1i, bl1, wl2, bl2,
              wr1u, wr1i, br1, wr2, br2,
              w3cat, b3cat, out_ref):
    u = users_ref[...]
    it = items_ref[...]

    def unpack(words):
        hi = lax.bitcast_convert_type(words & jnp.uint32(0xFFFF0000),
                                      jnp.float32)
        lo = lax.bitcast_convert_type(words << 16, jnp.float32)
        return jnp.concatenate([hi, lo], axis=1)

    ubits = lax.bitcast_convert_type(uraw_ref[...], jnp.uint32)
    ibits = lax.bitcast_convert_type(iraw_ref[...], jnp.uint32)
    s_u = ((u >> 11) & 3)[:, None]
    uw = jnp.where(
        s_u == 0, ubits[:, 0:32],
        jnp.where(s_u == 1, ubits[:, 32:64],
                  jnp.where(s_u == 2, ubits[:, 64:96], ubits[:, 96:128])))
    xu = unpack(uw)
    s_i = ((it >> 11) & 7)[:, None]
    iw = ibits[:, 112:128]
    for s in range(6, -1, -1):
        iw = jnp.where(s_i == s, ibits[:, 16 * s:16 * s + 16], iw)
    xi = unpack(iw)[:, 0:24]

    def head(w1u, w1i, b1, w2, b2):
        h = lax.dot(xu, w1u[...], preferred_element_type=jnp.float32)
        h = h + lax.dot(xi, w1i[...], preferred_element_type=jnp.float32)
        h = jnp.maximum(h + b1[...], 0.0)
        return jnp.maximum(
            lax.dot(h, w2[...], preferred_element_type=jnp.float32) + b2[...],
            0.0)

    hcat = jnp.concatenate([head(wl1u, wl1i, bl1, wl2, bl2),
                            head(wr1u, wr1i, br1, wr2, br2)], axis=1)
    z = lax.dot(hcat, w3cat[...],
                preferred_element_type=jnp.float32) + b3cat[...]
    out_ref[...] = 1.0 / (1.0 + jnp.exp(-z))


@functools.lru_cache(maxsize=None)
def _make_mlp(B, D_u, D_i, H1, H2):
    BLK = 2048
    grid = B // BLK

    def w_spec(shape):
        return pl.BlockSpec(shape, lambda i: (0,) * len(shape))

    return pl.pallas_call(
        _mlp_body,
        grid=(grid,),
        in_specs=[
            pl.BlockSpec((BLK, 128), lambda i: (i, 0)),
            pl.BlockSpec((BLK, 128), lambda i: (i, 0)),
            pl.BlockSpec((BLK,), lambda i: (i,)),
            pl.BlockSpec((BLK,), lambda i: (i,)),
            w_spec((D_u, H1)), w_spec((D_i, H1)), w_spec((1, H1)),
            w_spec((H1, H2)), w_spec((1, H2)),
            w_spec((D_u, H1)), w_spec((D_i, H1)), w_spec((1, H1)),
            w_spec((H1, H2)), w_spec((1, H2)),
            w_spec((2 * H2, 2)), w_spec((1, 2)),
        ],
        out_specs=pl.BlockSpec((BLK, 2), lambda i: (i, 0)),
        out_shape=jax.ShapeDtypeStruct((B, 2), jnp.float32),
    )


def kernel(users, items, user_embedding, item_embedding,
           Wl1, bl1, Wl2, bl2, Wl3, bl3,
           Wr1, br1, Wr2, br2, Wr3, br3):
    B = users.shape[0]
    V_u, D_u = user_embedding.shape
    V_i, D_i = item_embedding.shape
    H1 = Wl1.shape[1]
    H2 = Wl2.shape[1]

    su = _make_pack(V_u, D_u, 4 * _ROWS, _pack_u_body)(user_embedding.T)
    si = _make_pack(V_i, D_i, 8 * _ROWS, _pack_i_body)(item_embedding.T)

    uraw = _make_gather(B, 13)(su, users)
    iraw = _make_gather(B, 14)(si, items)

    z16 = jnp.zeros((H2, 1), jnp.float32)
    w3cat = jnp.concatenate(
        [jnp.concatenate([Wl3, z16], axis=1),
         jnp.concatenate([z16, Wr3], axis=1)], axis=0)
    b3cat = jnp.concatenate([bl3, br3]).reshape(1, 2)

    mlp = _make_mlp(B, D_u, D_i, H1, H2)
    out = mlp(
        uraw, iraw, users, items,
        Wl1[:D_u], Wl1[D_u:], bl1.reshape(1, H1),
        Wl2, bl2.reshape(1, H2),
        Wr1[:D_u], Wr1[D_u:], br1.reshape(1, H1),
        Wr2, br2.reshape(1, H2),
        w3cat, b3cat,
    )
    return out[:, 0], out[:, 1]


# speedup vs baseline: 5.0809x; 1.1848x over previous
import functools

import jax
import jax.numpy as jnp
from jax import lax
from jax.experimental import pallas as pl
from jax.experimental.pallas import tpu as pltpu
from jax.experimental.pallas import tpu_sc as plsc

_NW = 32
_IDX_CHUNK = 128


def _round_bits(x):
    return lax.bitcast_convert_type(
        x.astype(jnp.bfloat16).astype(jnp.float32), jnp.uint32)


_ROWS = 4096
_RS = 12


def _pack_u_body(x_ref, out_ref):
    xb = _round_bits(x_ref[...])
    w = xb[0:32] | (xb[32:64] >> 16)
    w2 = jnp.concatenate(
        [w[:, s * _ROWS:(s + 1) * _ROWS] for s in range(4)], axis=0)
    out_ref[...] = lax.bitcast_convert_type(w2, jnp.float32).T


def _pack_i_body(x_ref, out_ref):
    xb = _round_bits(x_ref[...])
    lo = jnp.concatenate(
        [xb[16:24] >> 16, jnp.zeros((8, xb.shape[1]), jnp.uint32)], axis=0)
    w = xb[0:16] | lo
    w2 = jnp.concatenate(
        [w[:, s * _ROWS:(s + 1) * _ROWS] for s in range(8)], axis=0)
    out_ref[...] = lax.bitcast_convert_type(w2, jnp.float32).T


@functools.lru_cache(maxsize=None)
def _make_pack(V, D, cols_per_block, body):
    n = pl.cdiv(V, cols_per_block)
    return pl.pallas_call(
        body,
        grid=(n,),
        in_specs=[pl.BlockSpec((D, cols_per_block), lambda i: (0, i))],
        out_specs=pl.BlockSpec((_ROWS, 128), lambda i: (i, 0)),
        out_shape=jax.ShapeDtypeStruct((n * _ROWS, 128), jnp.float32),
    )


@functools.lru_cache(maxsize=None)
def _make_gather(B, row_shift):
    b_per_w = B // _NW
    n_chunks = b_per_w // _IDX_CHUNK
    mesh = plsc.VectorSubcoreMesh(core_axis_name="c", subcore_axis_name="s")

    @functools.partial(
        pl.kernel,
        mesh=mesh,
        out_type=jax.ShapeDtypeStruct((B, 128), jnp.float32),
        scratch_types=[
            pltpu.VMEM((n_chunks, _IDX_CHUNK), jnp.int32),
            pltpu.VMEM((b_per_w, 128), jnp.float32),
            pltpu.SemaphoreType.DMA,
        ],
    )
    def gather_k(tab, idx_hbm, out, idx, rows, sem):
        wid = lax.axis_index("s") * 2 + lax.axis_index("c")
        base = wid * b_per_w
        for j in range(n_chunks):
            pltpu.sync_copy(idx_hbm.at[pl.ds(base + j * _IDX_CHUNK,
                                             _IDX_CHUNK)],
                            idx.at[j])
        for j in range(n_chunks):
            for k in range(_IDX_CHUNK // 16):
                s = pl.ds(k * 16, 16)
                v = idx[j, s]
                idx[j, s] = ((v >> row_shift) << _RS) | (v & (_ROWS - 1))
        copies = []
        for j in range(n_chunks):
            copies.append(pltpu.async_copy(
                tab.at[idx.at[j]],
                rows.at[pl.ds(j * _IDX_CHUNK, _IDX_CHUNK)], sem))
        for c in copies:
            c.wait()
        pltpu.sync_copy(rows, out.at[pl.ds(base, b_per_w)])

    return gather_k


def _mlp_body(uraw_ref, iraw_ref, users_ref, items_ref,
              wl1u, wl1i, bl1, wl2, bl2,
              wr1u, wr1i, br1, wr2, br2,
              w3cat, b3cat, out_ref):
    u = users_ref[...]
    it = items_ref[...]

    def unpack(words):
        hi = lax.bitcast_convert_type(words & jnp.uint32(0xFFFF0000),
                                      jnp.float32)
        lo = lax.bitcast_convert_type(words << 16, jnp.float32)
        return jnp.concatenate([hi, lo], axis=1)

    ubits = lax.bitcast_convert_type(uraw_ref[...], jnp.uint32)
    ibits = lax.bitcast_convert_type(iraw_ref[...], jnp.uint32)
    s_u = ((u >> _RS) & 3)[:, None]
    uw = jnp.where(
        s_u == 0, ubits[:, 0:32],
        jnp.where(s_u == 1, ubits[:, 32:64],
                  jnp.where(s_u == 2, ubits[:, 64:96], ubits[:, 96:128])))
    xu = unpack(uw)
    s_i = ((it >> _RS) & 7)[:, None]
    iw = ibits[:, 112:128]
    for s in range(6, -1, -1):
        iw = jnp.where(s_i == s, ibits[:, 16 * s:16 * s + 16], iw)
    xi = unpack(iw)[:, 0:24]

    def head(w1u, w1i, b1, w2, b2):
        h = lax.dot(xu, w1u[...], preferred_element_type=jnp.float32)
        h = h + lax.dot(xi, w1i[...], preferred_element_type=jnp.float32)
        h = jnp.maximum(h + b1[...], 0.0)
        return jnp.maximum(
            lax.dot(h, w2[...], preferred_element_type=jnp.float32) + b2[...],
            0.0)

    hcat = jnp.concatenate([head(wl1u, wl1i, bl1, wl2, bl2),
                            head(wr1u, wr1i, br1, wr2, br2)], axis=1)
    z = lax.dot(hcat, w3cat[...],
                preferred_element_type=jnp.float32) + b3cat[...]
    out_ref[...] = 1.0 / (1.0 + jnp.exp(-z))


@functools.lru_cache(maxsize=None)
def _make_mlp(B, D_u, D_i, H1, H2):
    BLK = 2048
    grid = B // BLK

    def w_spec(shape):
        return pl.BlockSpec(shape, lambda i: (0,) * len(shape))

    return pl.pallas_call(
        _mlp_body,
        grid=(grid,),
        in_specs=[
            pl.BlockSpec((BLK, 128), lambda i: (i, 0)),
            pl.BlockSpec((BLK, 128), lambda i: (i, 0)),
            pl.BlockSpec((BLK,), lambda i: (i,)),
            pl.BlockSpec((BLK,), lambda i: (i,)),
            w_spec((D_u, H1)), w_spec((D_i, H1)), w_spec((1, H1)),
            w_spec((H1, H2)), w_spec((1, H2)),
            w_spec((D_u, H1)), w_spec((D_i, H1)), w_spec((1, H1)),
            w_spec((H1, H2)), w_spec((1, H2)),
            w_spec((2 * H2, 2)), w_spec((1, 2)),
        ],
        out_specs=pl.BlockSpec((BLK, 2), lambda i: (i, 0)),
        out_shape=jax.ShapeDtypeStruct((B, 2), jnp.float32),
    )


def kernel(users, items, user_embedding, item_embedding,
           Wl1, bl1, Wl2, bl2, Wl3, bl3,
           Wr1, br1, Wr2, br2, Wr3, br3):
    B = users.shape[0]
    V_u, D_u = user_embedding.shape
    V_i, D_i = item_embedding.shape
    H1 = Wl1.shape[1]
    H2 = Wl2.shape[1]

    su = _make_pack(V_u, D_u, 4 * _ROWS, _pack_u_body)(user_embedding.T)
    si = _make_pack(V_i, D_i, 8 * _ROWS, _pack_i_body)(item_embedding.T)

    uraw = _make_gather(B, _RS + 2)(su, users)
    iraw = _make_gather(B, _RS + 3)(si, items)

    z16 = jnp.zeros((H2, 1), jnp.float32)
    w3cat = jnp.concatenate(
        [jnp.concatenate([Wl3, z16], axis=1),
         jnp.concatenate([z16, Wr3], axis=1)], axis=0)
    b3cat = jnp.concatenate([bl3, br3]).reshape(1, 2)

    mlp = _make_mlp(B, D_u, D_i, H1, H2)
    out = mlp(
        uraw, iraw, users, items,
        Wl1[:D_u], Wl1[D_u:], bl1.reshape(1, H1),
        Wl2, bl2.reshape(1, H2),
        Wr1[:D_u], Wr1[D_u:], br1.reshape(1, H1),
        Wr2, br2.reshape(1, H2),
        w3cat, b3cat,
    )
    return out[:, 0], out[:, 1]


# speedup vs baseline: 5.2567x; 1.0346x over previous
import functools

import jax
import jax.numpy as jnp
from jax import lax
from jax.experimental import pallas as pl
from jax.experimental.pallas import tpu as pltpu
from jax.experimental.pallas import tpu_sc as plsc

_NW = 32
_IDX_CHUNK = 128


def _round_bits(x):
    return lax.bitcast_convert_type(
        x.astype(jnp.bfloat16).astype(jnp.float32), jnp.uint32)


_ROWS = 8192
_RS = 13


def _pack_u_body(x_ref, out_ref):
    xb = _round_bits(x_ref[...])
    w = xb[0:32] | (xb[32:64] >> 16)
    w2 = jnp.concatenate(
        [w[:, s * _ROWS:(s + 1) * _ROWS] for s in range(4)], axis=0)
    out_ref[...] = lax.bitcast_convert_type(w2, jnp.float32).T


def _pack_i_body(x_ref, out_ref):
    xb = _round_bits(x_ref[...])
    lo = jnp.concatenate(
        [xb[16:24] >> 16, jnp.zeros((8, xb.shape[1]), jnp.uint32)], axis=0)
    w = xb[0:16] | lo
    w2 = jnp.concatenate(
        [w[:, s * _ROWS:(s + 1) * _ROWS] for s in range(8)], axis=0)
    out_ref[...] = lax.bitcast_convert_type(w2, jnp.float32).T


@functools.lru_cache(maxsize=None)
def _make_pack(V, D, cols_per_block, body):
    n = pl.cdiv(V, cols_per_block)
    return pl.pallas_call(
        body,
        grid=(n,),
        in_specs=[pl.BlockSpec((D, cols_per_block), lambda i: (0, i))],
        out_specs=pl.BlockSpec((_ROWS, 128), lambda i: (i, 0)),
        out_shape=jax.ShapeDtypeStruct((n * _ROWS, 128), jnp.float32),
    )


@functools.lru_cache(maxsize=None)
def _make_gather(B, row_shift):
    b_per_w = B // _NW
    n_chunks = b_per_w // _IDX_CHUNK
    mesh = plsc.VectorSubcoreMesh(core_axis_name="c", subcore_axis_name="s")

    @functools.partial(
        pl.kernel,
        mesh=mesh,
        out_type=jax.ShapeDtypeStruct((B, 128), jnp.float32),
        scratch_types=[
            pltpu.VMEM((n_chunks, _IDX_CHUNK), jnp.int32),
            pltpu.VMEM((b_per_w, 128), jnp.float32),
            pltpu.SemaphoreType.DMA,
        ],
    )
    def gather_k(tab, idx_hbm, out, idx, rows, sem):
        wid = lax.axis_index("s") * 2 + lax.axis_index("c")
        base = wid * b_per_w
        for j in range(n_chunks):
            pltpu.sync_copy(idx_hbm.at[pl.ds(base + j * _IDX_CHUNK,
                                             _IDX_CHUNK)],
                            idx.at[j])
        for j in range(n_chunks):
            for k in range(_IDX_CHUNK // 16):
                s = pl.ds(k * 16, 16)
                v = idx[j, s]
                idx[j, s] = ((v >> row_shift) << _RS) | (v & (_ROWS - 1))
        copies = []
        for j in range(n_chunks):
            copies.append(pltpu.async_copy(
                tab.at[idx.at[j]],
                rows.at[pl.ds(j * _IDX_CHUNK, _IDX_CHUNK)], sem))
        for c in copies:
            c.wait()
        pltpu.sync_copy(rows, out.at[pl.ds(base, b_per_w)])

    return gather_k


def _mlp_body(uraw_ref, iraw_ref, users_ref, items_ref,
              wl1u, wl1i, bl1, wl2, bl2,
              wr1u, wr1i, br1, wr2, br2,
              w3cat, b3cat, out_ref):
    u = users_ref[...]
    it = items_ref[...]

    def unpack(words):
        hi = lax.bitcast_convert_type(words & jnp.uint32(0xFFFF0000),
                                      jnp.float32)
        lo = lax.bitcast_convert_type(words << 16, jnp.float32)
        return jnp.concatenate([hi, lo], axis=1)

    ubits = lax.bitcast_convert_type(uraw_ref[...], jnp.uint32)
    ibits = lax.bitcast_convert_type(iraw_ref[...], jnp.uint32)
    s_u = ((u >> _RS) & 3)[:, None]
    uw = jnp.where(
        s_u == 0, ubits[:, 0:32],
        jnp.where(s_u == 1, ubits[:, 32:64],
                  jnp.where(s_u == 2, ubits[:, 64:96], ubits[:, 96:128])))
    xu = unpack(uw)
    s_i = ((it >> _RS) & 7)[:, None]
    iw = ibits[:, 112:128]
    for s in range(6, -1, -1):
        iw = jnp.where(s_i == s, ibits[:, 16 * s:16 * s + 16], iw)
    xi = unpack(iw)[:, 0:24]

    def head(w1u, w1i, b1, w2, b2):
        h = lax.dot(xu, w1u[...], preferred_element_type=jnp.float32)
        h = h + lax.dot(xi, w1i[...], preferred_element_type=jnp.float32)
        h = jnp.maximum(h + b1[...], 0.0)
        return jnp.maximum(
            lax.dot(h, w2[...], preferred_element_type=jnp.float32) + b2[...],
            0.0)

    hcat = jnp.concatenate([head(wl1u, wl1i, bl1, wl2, bl2),
                            head(wr1u, wr1i, br1, wr2, br2)], axis=1)
    z = lax.dot(hcat, w3cat[...],
                preferred_element_type=jnp.float32) + b3cat[...]
    out_ref[...] = 1.0 / (1.0 + jnp.exp(-z))


@functools.lru_cache(maxsize=None)
def _make_mlp(B, D_u, D_i, H1, H2):
    BLK = 2048
    grid = B // BLK

    def w_spec(shape):
        return pl.BlockSpec(shape, lambda i: (0,) * len(shape))

    return pl.pallas_call(
        _mlp_body,
        grid=(grid,),
        in_specs=[
            pl.BlockSpec((BLK, 128), lambda i: (i, 0)),
            pl.BlockSpec((BLK, 128), lambda i: (i, 0)),
            pl.BlockSpec((BLK,), lambda i: (i,)),
            pl.BlockSpec((BLK,), lambda i: (i,)),
            w_spec((D_u, H1)), w_spec((D_i, H1)), w_spec((1, H1)),
            w_spec((H1, H2)), w_spec((1, H2)),
            w_spec((D_u, H1)), w_spec((D_i, H1)), w_spec((1, H1)),
            w_spec((H1, H2)), w_spec((1, H2)),
            w_spec((2 * H2, 2)), w_spec((1, 2)),
        ],
        out_specs=pl.BlockSpec((BLK, 2), lambda i: (i, 0)),
        out_shape=jax.ShapeDtypeStruct((B, 2), jnp.float32),
    )


def kernel(users, items, user_embedding, item_embedding,
           Wl1, bl1, Wl2, bl2, Wl3, bl3,
           Wr1, br1, Wr2, br2, Wr3, br3):
    B = users.shape[0]
    V_u, D_u = user_embedding.shape
    V_i, D_i = item_embedding.shape
    H1 = Wl1.shape[1]
    H2 = Wl2.shape[1]

    su = _make_pack(V_u, D_u, 4 * _ROWS, _pack_u_body)(user_embedding.T)
    si = _make_pack(V_i, D_i, 8 * _ROWS, _pack_i_body)(item_embedding.T)

    uraw = _make_gather(B, _RS + 2)(su, users)
    iraw = _make_gather(B, _RS + 3)(si, items)

    z16 = jnp.zeros((H2, 1), jnp.float32)
    w3cat = jnp.concatenate(
        [jnp.concatenate([Wl3, z16], axis=1),
         jnp.concatenate([z16, Wr3], axis=1)], axis=0)
    b3cat = jnp.concatenate([bl3, br3]).reshape(1, 2)

    mlp = _make_mlp(B, D_u, D_i, H1, H2)
    out = mlp(
        uraw, iraw, users, items,
        Wl1[:D_u], Wl1[D_u:], bl1.reshape(1, H1),
        Wl2, bl2.reshape(1, H2),
        Wr1[:D_u], Wr1[D_u:], br1.reshape(1, H1),
        Wr2, br2.reshape(1, H2),
        w3cat, b3cat,
    )
    return out[:, 0], out[:, 1]
